# Initial kernel scaffold; baseline (speedup 1.0000x reference)
import jax
import jax.numpy as jnp
from jax.experimental import pallas as pl


def kernel(action_indices, embedding_weight):
    raise NotImplementedError("write your pallas kernel here")


# speedup vs baseline: 1.8709x; 1.8709x over previous
import functools

import jax
import jax.numpy as jnp
from jax import lax
from jax.experimental import pallas as pl
from jax.experimental.pallas import tpu as pltpu
from jax.experimental.pallas import tpu_sc as plsc

ACTION_SIZE = 1000000
EMBED_DIM = 64

NC = 2
NS = 16
NW = NC * NS

B_TOTAL = 16384 * 50
B_PER_W = B_TOTAL // NW
BLK = 128
NBLK = B_PER_W // BLK
NBUF = 4
NGRP = NBLK // NBUF


def _gather_body(idx_hbm, table_hbm, out_hbm, idx_v, rows_v, gsem, wsem):
    wid = lax.axis_index("s") * NC + lax.axis_index("c")
    base = wid * B_PER_W

    pltpu.sync_copy(idx_hbm.at[wid], idx_v)

    def gather_start(j, b):
        pltpu.make_async_copy(
            table_hbm.at[idx_v.at[j]], rows_v.at[b], gsem.at[b]
        ).start()

    def gather_wait(j, b):
        pltpu.make_async_copy(
            table_hbm.at[idx_v.at[j]], rows_v.at[b], gsem.at[b]
        ).wait()

    def write_start(j, b):
        pltpu.make_async_copy(
            rows_v.at[b], out_hbm.at[pl.ds(base + j * BLK, BLK)], wsem.at[b]
        ).start()

    def write_wait(j, b):
        pltpu.make_async_copy(
            rows_v.at[b], out_hbm.at[pl.ds(base + j * BLK, BLK)], wsem.at[b]
        ).wait()

    for b in range(NBUF):
        gather_start(b, b)

    def group_body(g, carry):
        j0 = g * NBUF
        for b in range(NBUF):
            gather_wait(j0 + b, b)
            write_start(j0 + b, b)
        for b in range(NBUF):
            write_wait(j0 + b, b)
            gather_start(j0 + NBUF + b, b)
        return carry

    lax.fori_loop(0, NGRP - 1, group_body, 0)

    j0 = (NGRP - 1) * NBUF
    for b in range(NBUF):
        gather_wait(j0 + b, b)
        write_start(j0 + b, b)
    for b in range(NBUF):
        write_wait(j0 + b, b)


@jax.jit
def _embedding_gather(idx3, table):
    mesh = plsc.VectorSubcoreMesh(core_axis_name="c", subcore_axis_name="s")
    run = functools.partial(
        pl.kernel,
        out_type=jax.ShapeDtypeStruct((B_TOTAL, EMBED_DIM), jnp.float32),
        mesh=mesh,
        scratch_types=[
            pltpu.VMEM((NBLK, BLK), jnp.int32),
            pltpu.VMEM((NBUF, BLK, EMBED_DIM), jnp.float32),
            pltpu.SemaphoreType.DMA((NBUF,)),
            pltpu.SemaphoreType.DMA((NBUF,)),
        ],
        compiler_params=pltpu.CompilerParams(use_tc_tiling_on_sc=False),
    )(_gather_body)
    return run(idx3, table)


def kernel(action_indices, embedding_weight):
    idx3 = jnp.asarray(action_indices, jnp.int32).reshape(NW, NBLK, BLK)
    out = _embedding_gather(idx3, embedding_weight)
    return out.reshape(action_indices.shape + (EMBED_DIM,))
